# baseline (speedup 1.0000x reference)
import jax
import jax.numpy as jnp
from jax import lax
from jax.experimental import pallas as pl
from jax.experimental.pallas import tpu as pltpu
from jax.experimental.pallas import tpu_sc as plsc

N, L, D, NBINS = 4096, 64, 32, 256
NC, NS = 2, 16
NW = NC * NS
CPW = (L * D) // NW
G = CPW // 16
HPW = CPW * NBINS
CHUNK = 256
NCHUNK = N // CHUNK

_MESH = plsc.VectorSubcoreMesh(
    core_axis_name="c", subcore_axis_name="s", num_cores=NC, num_subcores=NS
)
_PARAMS = pltpu.CompilerParams(needs_layout_passes=False)


def _worker_id():
    return lax.axis_index("c") * NS + lax.axis_index("s")


def _double_buffered(src_hbm, wid, bufs, sems, consume):
    copies = [None, None]
    copies[0] = pltpu.async_copy(
        src_hbm.at[pl.ds(0, CHUNK), wid], bufs[0], sems[0]
    )
    for ch in range(NCHUNK):
        cur = ch % 2
        copies[cur].wait()
        if ch + 1 < NCHUNK:
            nxt = 1 - cur
            copies[nxt] = pltpu.async_copy(
                src_hbm.at[pl.ds((ch + 1) * CHUNK, CHUNK), wid],
                bufs[nxt],
                sems[nxt],
            )
        consume(bufs[cur])


def _hist_pass(bufs, sems, src_hbm, wid, hist, lo, hi, scale, base):
    ones = jnp.ones((16,), jnp.float32)

    def consume(buf):
        @plsc.parallel_loop(0, CHUNK, unroll=3)
        def body(i):
            for g in range(G):
                x = buf[i, pl.ds(g * 16, 16)]
                tb = (x - lo[g]) * scale[g]
                tb = jnp.minimum(jnp.maximum(tb, 0.0), jnp.float32(NBINS - 1))
                idx = tb.astype(jnp.int32)
                within = (x >= lo[g]) & (x <= hi[g])
                plsc.addupdate_scatter(hist, [idx + base[g]], ones,
                                       mask=within)

    _double_buffered(src_hbm, wid, bufs, sems, consume)


def _real_body(xr_hbm, hist_hbm, lo_hbm, hi_hbm, buf0, buf1, hist_r,
               stage_v, sem0, sem1):
    wid = _worker_id()
    iota = lax.iota(jnp.int32, 16)
    base = [(g * 16 + iota) * NBINS for g in range(G)]
    bufs, sems = [buf0, buf1], [sem0, sem1]
    zeros = jnp.zeros((16,), jnp.float32)

    carry0 = tuple(
        tuple(jnp.full((16,), s * jnp.inf, jnp.float32) for _ in range(G))
        for s in (1.0, -1.0, 1.0, -1.0)
    )
    state = [carry0]

    def mm_consume(buf):
        def mmbody(i, carry):
            mn_a, mx_a, mn_b, mx_b = carry
            new_mn_a = tuple(
                jnp.minimum(mn_a[g], buf[i, pl.ds(g * 16, 16)])
                for g in range(G)
            )
            new_mx_a = tuple(
                jnp.maximum(mx_a[g], buf[i, pl.ds(g * 16, 16)])
                for g in range(G)
            )
            new_mn_b = tuple(
                jnp.minimum(mn_b[g], buf[i + 1, pl.ds(g * 16, 16)])
                for g in range(G)
            )
            new_mx_b = tuple(
                jnp.maximum(mx_b[g], buf[i + 1, pl.ds(g * 16, 16)])
                for g in range(G)
            )
            return new_mn_a, new_mx_a, new_mn_b, new_mx_b

        state[0] = plsc.parallel_loop(0, CHUNK, step=2, unroll=2,
                                      carry=state[0])(mmbody)

    _double_buffered(xr_hbm, wid, bufs, sems, mm_consume)
    mn_a, mx_a, mn_b, mx_b = state[0]
    mns = [jnp.minimum(mn_a[g], mn_b[g]) for g in range(G)]
    mxs = [jnp.maximum(mx_a[g], mx_b[g]) for g in range(G)]

    lo, hi, scale = [], [], []
    for g in range(G):
        mn, mx = mns[g], mxs[g]
        same = jnp.abs(mx - mn) < 1e-10
        mx = jnp.where(same, mx + 1e-5, mx)
        mn = jnp.where(same, mn - 1e-5, mn)
        lo.append(mn)
        hi.append(mx)
        scale.append((1.0 / (mx - mn)) * jnp.float32(NBINS))

    for g in range(G):
        stage_v[pl.ds(g * 16, 16)] = lo[g]
    pltpu.sync_copy(stage_v, lo_hbm.at[pl.ds(wid * CPW, CPW)])
    for g in range(G):
        stage_v[pl.ds(g * 16, 16)] = hi[g]
    pltpu.sync_copy(stage_v, hi_hbm.at[pl.ds(wid * CPW, CPW)])

    @plsc.parallel_loop(0, HPW // 16, unroll=4)
    def zbody(i):
        hist_r[pl.ds(i * 16, 16)] = zeros

    _hist_pass(bufs, sems, xr_hbm, wid, hist_r, lo, hi, scale, base)
    pltpu.sync_copy(hist_r, hist_hbm.at[pl.ds(wid * HPW, HPW)])


def _fake_body(xf_hbm, hist_hbm, lo_hbm, hi_hbm, out_hbm, buf0, buf1,
               hist_r, hist_f, stage_v, sem0, sem1):
    wid = _worker_id()
    iota = lax.iota(jnp.int32, 16)
    base = [(g * 16 + iota) * NBINS for g in range(G)]
    bufs, sems = [buf0, buf1], [sem0, sem1]
    zeros = jnp.zeros((16,), jnp.float32)

    hist_copy = pltpu.async_copy(
        hist_hbm.at[pl.ds(wid * HPW, HPW)], hist_r, sem1
    )
    pltpu.sync_copy(lo_hbm.at[pl.ds(wid * CPW, CPW)], stage_v)
    lo = [stage_v[pl.ds(g * 16, 16)] for g in range(G)]
    pltpu.sync_copy(hi_hbm.at[pl.ds(wid * CPW, CPW)], stage_v)
    hi = [stage_v[pl.ds(g * 16, 16)] for g in range(G)]
    scale = [(1.0 / (hi[g] - lo[g])) * jnp.float32(NBINS) for g in range(G)]
    hist_copy.wait()

    @plsc.parallel_loop(0, HPW // 16, unroll=4)
    def zbody(i):
        hist_f[pl.ds(i * 16, 16)] = zeros

    _hist_pass(bufs, sems, xf_hbm, wid, hist_f, lo, hi, scale, base)

    inv_n = jnp.float32(1.0 / N)
    for g in range(G):
        colbase = base[g]

        def fbody(b, carry):
            sa, sf = carry
            cr = plsc.load_gather(hist_r, [colbase + b])
            cf = plsc.load_gather(hist_f, [colbase + b])
            return sa + jnp.abs(cf - cr), sf + cf

        sa, sf = plsc.parallel_loop(0, NBINS, unroll=4, carry=(zeros, zeros))(
            fbody
        )
        loss_g = sa * inv_n + (jnp.float32(N) - sf) * inv_n
        bw = (hi[g] - lo[g]) * jnp.float32(1.0 / NBINS)
        c_first = lo[g] + bw * jnp.float32(0.5)
        c_last = lo[g] + bw * jnp.float32(NBINS - 0.5)
        deg = (jnp.abs(c_first) < 1e-16) & (jnp.abs(c_last) < 1e-16)
        loss_g = jnp.where(deg, jnp.float32(2.0), loss_g)
        stage_v[pl.ds(g * 16, 16)] = loss_g

    pltpu.sync_copy(stage_v, out_hbm.at[pl.ds(wid * CPW, CPW)])


@jax.jit
def _hist_loss(xr, xf):
    f32 = jnp.float32
    hist_hbm, lo_hbm, hi_hbm = pl.kernel(
        _real_body,
        out_type=(
            jax.ShapeDtypeStruct((NW * HPW,), f32),
            jax.ShapeDtypeStruct((NW * CPW,), f32),
            jax.ShapeDtypeStruct((NW * CPW,), f32),
        ),
        mesh=_MESH,
        compiler_params=_PARAMS,
        scratch_types=[
            pltpu.VMEM((CHUNK, CPW), f32),
            pltpu.VMEM((CHUNK, CPW), f32),
            pltpu.VMEM((HPW,), f32),
            pltpu.VMEM((CPW,), f32),
            pltpu.SemaphoreType.DMA,
            pltpu.SemaphoreType.DMA,
        ],
    )(xr)
    return pl.kernel(
        _fake_body,
        out_type=jax.ShapeDtypeStruct((L * D,), f32),
        mesh=_MESH,
        compiler_params=_PARAMS,
        scratch_types=[
            pltpu.VMEM((CHUNK, CPW), f32),
            pltpu.VMEM((CHUNK, CPW), f32),
            pltpu.VMEM((HPW,), f32),
            pltpu.VMEM((HPW,), f32),
            pltpu.VMEM((CPW,), f32),
            pltpu.SemaphoreType.DMA,
            pltpu.SemaphoreType.DMA,
        ],
    )(xf, hist_hbm, lo_hbm, hi_hbm)


def kernel(x_real, x_fake, n_bins):
    del n_bins
    xr = x_real.reshape(N, NW, CPW)
    xf = x_fake.reshape(N, NW, CPW)
    return _hist_loss(xr, xf).reshape(L, D)


# speedup vs baseline: 1.0088x; 1.0088x over previous
import jax
import jax.numpy as jnp
from jax import lax
from jax.experimental import pallas as pl
from jax.experimental.pallas import tpu as pltpu
from jax.experimental.pallas import tpu_sc as plsc

N, L, D, NBINS = 4096, 64, 32, 256
NC, NS = 2, 16
NW = NC * NS
CPW = (L * D) // NW
G = CPW // 16
HPW = CPW * NBINS
CHUNK = 256
NCHUNK = N // CHUNK

_MESH = plsc.VectorSubcoreMesh(
    core_axis_name="c", subcore_axis_name="s", num_cores=NC, num_subcores=NS
)
_PARAMS = pltpu.CompilerParams(needs_layout_passes=False)


def _worker_id():
    return lax.axis_index("c") * NS + lax.axis_index("s")


def _double_buffered(src_hbm, wid, bufs, sems, consume):
    copies = [None, None]
    copies[0] = pltpu.async_copy(
        src_hbm.at[pl.ds(0, CHUNK), wid], bufs[0], sems[0]
    )
    for ch in range(NCHUNK):
        cur = ch % 2
        copies[cur].wait()
        if ch + 1 < NCHUNK:
            nxt = 1 - cur
            copies[nxt] = pltpu.async_copy(
                src_hbm.at[pl.ds((ch + 1) * CHUNK, CHUNK), wid],
                bufs[nxt],
                sems[nxt],
            )
        consume(bufs[cur])


def _hist_pass(bufs, sems, src_hbm, wid, hist, lo, hi, scale, base):
    ones = jnp.ones((16,), jnp.float32)

    def consume(buf):
        @plsc.parallel_loop(0, CHUNK, unroll=2)
        def body(i):
            for g in range(G):
                x = buf[i, pl.ds(g * 16, 16)]
                tb = (x - lo[g]) * scale[g]
                tb = jnp.minimum(jnp.maximum(tb, 0.0), jnp.float32(NBINS - 1))
                idx = tb.astype(jnp.int32)
                within = (x >= lo[g]) & (x <= hi[g])
                plsc.addupdate_scatter(hist, [idx + base[g]], ones,
                                       mask=within)

    _double_buffered(src_hbm, wid, bufs, sems, consume)


def _real_body(xr_hbm, hist_hbm, lo_hbm, hi_hbm, buf0, buf1, hist_r,
               stage_v, sem0, sem1):
    wid = _worker_id()
    iota = lax.iota(jnp.int32, 16)
    base = [(g * 16 + iota) * NBINS for g in range(G)]
    bufs, sems = [buf0, buf1], [sem0, sem1]
    zeros = jnp.zeros((16,), jnp.float32)

    carry0 = tuple(
        tuple(jnp.full((16,), s * jnp.inf, jnp.float32) for _ in range(G))
        for s in (1.0, -1.0, 1.0, -1.0)
    )
    state = [carry0]

    def mm_consume(buf):
        def mmbody(i, carry):
            mn_a, mx_a, mn_b, mx_b = carry
            new_mn_a = tuple(
                jnp.minimum(mn_a[g], buf[i, pl.ds(g * 16, 16)])
                for g in range(G)
            )
            new_mx_a = tuple(
                jnp.maximum(mx_a[g], buf[i, pl.ds(g * 16, 16)])
                for g in range(G)
            )
            new_mn_b = tuple(
                jnp.minimum(mn_b[g], buf[i + 1, pl.ds(g * 16, 16)])
                for g in range(G)
            )
            new_mx_b = tuple(
                jnp.maximum(mx_b[g], buf[i + 1, pl.ds(g * 16, 16)])
                for g in range(G)
            )
            return new_mn_a, new_mx_a, new_mn_b, new_mx_b

        state[0] = plsc.parallel_loop(0, CHUNK, step=2, unroll=2,
                                      carry=state[0])(mmbody)

    _double_buffered(xr_hbm, wid, bufs, sems, mm_consume)
    mn_a, mx_a, mn_b, mx_b = state[0]
    mns = [jnp.minimum(mn_a[g], mn_b[g]) for g in range(G)]
    mxs = [jnp.maximum(mx_a[g], mx_b[g]) for g in range(G)]

    lo, hi, scale = [], [], []
    for g in range(G):
        mn, mx = mns[g], mxs[g]
        same = jnp.abs(mx - mn) < 1e-10
        mx = jnp.where(same, mx + 1e-5, mx)
        mn = jnp.where(same, mn - 1e-5, mn)
        lo.append(mn)
        hi.append(mx)
        scale.append((1.0 / (mx - mn)) * jnp.float32(NBINS))

    for g in range(G):
        stage_v[pl.ds(g * 16, 16)] = lo[g]
    pltpu.sync_copy(stage_v, lo_hbm.at[pl.ds(wid * CPW, CPW)])
    for g in range(G):
        stage_v[pl.ds(g * 16, 16)] = hi[g]
    pltpu.sync_copy(stage_v, hi_hbm.at[pl.ds(wid * CPW, CPW)])

    @plsc.parallel_loop(0, HPW // 16, unroll=4)
    def zbody(i):
        hist_r[pl.ds(i * 16, 16)] = zeros

    _hist_pass(bufs, sems, xr_hbm, wid, hist_r, lo, hi, scale, base)
    pltpu.sync_copy(hist_r, hist_hbm.at[pl.ds(wid * HPW, HPW)])


def _fake_body(xf_hbm, hist_hbm, lo_hbm, hi_hbm, out_hbm, buf0, buf1,
               hist_r, hist_f, stage_v, sem0, sem1):
    wid = _worker_id()
    iota = lax.iota(jnp.int32, 16)
    base = [(g * 16 + iota) * NBINS for g in range(G)]
    bufs, sems = [buf0, buf1], [sem0, sem1]
    zeros = jnp.zeros((16,), jnp.float32)

    hist_copy = pltpu.async_copy(
        hist_hbm.at[pl.ds(wid * HPW, HPW)], hist_r, sem1
    )
    pltpu.sync_copy(lo_hbm.at[pl.ds(wid * CPW, CPW)], stage_v)
    lo = [stage_v[pl.ds(g * 16, 16)] for g in range(G)]
    pltpu.sync_copy(hi_hbm.at[pl.ds(wid * CPW, CPW)], stage_v)
    hi = [stage_v[pl.ds(g * 16, 16)] for g in range(G)]
    scale = [(1.0 / (hi[g] - lo[g])) * jnp.float32(NBINS) for g in range(G)]
    hist_copy.wait()

    @plsc.parallel_loop(0, HPW // 16, unroll=4)
    def zbody(i):
        hist_f[pl.ds(i * 16, 16)] = zeros

    _hist_pass(bufs, sems, xf_hbm, wid, hist_f, lo, hi, scale, base)

    inv_n = jnp.float32(1.0 / N)
    for g in range(G):
        colbase = base[g]

        def fbody(b, carry):
            sa, sf = carry
            cr = plsc.load_gather(hist_r, [colbase + b])
            cf = plsc.load_gather(hist_f, [colbase + b])
            return sa + jnp.abs(cf - cr), sf + cf

        sa, sf = plsc.parallel_loop(0, NBINS, unroll=4, carry=(zeros, zeros))(
            fbody
        )
        loss_g = sa * inv_n + (jnp.float32(N) - sf) * inv_n
        bw = (hi[g] - lo[g]) * jnp.float32(1.0 / NBINS)
        c_first = lo[g] + bw * jnp.float32(0.5)
        c_last = lo[g] + bw * jnp.float32(NBINS - 0.5)
        deg = (jnp.abs(c_first) < 1e-16) & (jnp.abs(c_last) < 1e-16)
        loss_g = jnp.where(deg, jnp.float32(2.0), loss_g)
        stage_v[pl.ds(g * 16, 16)] = loss_g

    pltpu.sync_copy(stage_v, out_hbm.at[pl.ds(wid * CPW, CPW)])


@jax.jit
def _hist_loss(xr, xf):
    f32 = jnp.float32
    hist_hbm, lo_hbm, hi_hbm = pl.kernel(
        _real_body,
        out_type=(
            jax.ShapeDtypeStruct((NW * HPW,), f32),
            jax.ShapeDtypeStruct((NW * CPW,), f32),
            jax.ShapeDtypeStruct((NW * CPW,), f32),
        ),
        mesh=_MESH,
        compiler_params=_PARAMS,
        scratch_types=[
            pltpu.VMEM((CHUNK, CPW), f32),
            pltpu.VMEM((CHUNK, CPW), f32),
            pltpu.VMEM((HPW,), f32),
            pltpu.VMEM((CPW,), f32),
            pltpu.SemaphoreType.DMA,
            pltpu.SemaphoreType.DMA,
        ],
    )(xr)
    return pl.kernel(
        _fake_body,
        out_type=jax.ShapeDtypeStruct((L * D,), f32),
        mesh=_MESH,
        compiler_params=_PARAMS,
        scratch_types=[
            pltpu.VMEM((CHUNK, CPW), f32),
            pltpu.VMEM((CHUNK, CPW), f32),
            pltpu.VMEM((HPW,), f32),
            pltpu.VMEM((HPW,), f32),
            pltpu.VMEM((CPW,), f32),
            pltpu.SemaphoreType.DMA,
            pltpu.SemaphoreType.DMA,
        ],
    )(xf, hist_hbm, lo_hbm, hi_hbm)


def kernel(x_real, x_fake, n_bins):
    del n_bins
    xr = x_real.reshape(N, NW, CPW)
    xf = x_fake.reshape(N, NW, CPW)
    return _hist_loss(xr, xf).reshape(L, D)


# speedup vs baseline: 1.0713x; 1.0620x over previous
import jax
import jax.numpy as jnp
from jax import lax
from jax.experimental import pallas as pl
from jax.experimental.pallas import tpu as pltpu
from jax.experimental.pallas import tpu_sc as plsc

N, L, D, NBINS = 4096, 64, 32, 256
NC, NS = 2, 16
NW = NC * NS
CPW = (L * D) // NW
G = CPW // 16
HPW = CPW * NBINS
CHUNK = 256
NCHUNK = N // CHUNK

_MESH = plsc.VectorSubcoreMesh(
    core_axis_name="c", subcore_axis_name="s", num_cores=NC, num_subcores=NS
)
_PARAMS = pltpu.CompilerParams(needs_layout_passes=False)


def _worker_id():
    return lax.axis_index("c") * NS + lax.axis_index("s")


def _double_buffered(src_hbm, wid, bufs, sems, consume):
    copies = [None, None]
    copies[0] = pltpu.async_copy(
        src_hbm.at[pl.ds(0, CHUNK), wid], bufs[0], sems[0]
    )
    for ch in range(NCHUNK):
        cur = ch % 2
        copies[cur].wait()
        if ch + 1 < NCHUNK:
            nxt = 1 - cur
            copies[nxt] = pltpu.async_copy(
                src_hbm.at[pl.ds((ch + 1) * CHUNK, CHUNK), wid],
                bufs[nxt],
                sems[nxt],
            )
        consume(bufs[cur])


def _hist_pass(bufs, sems, src_hbm, wid, hist, lo, hi, scale, base):
    ones = jnp.ones((16,), jnp.float32)

    def consume(buf):
        @plsc.parallel_loop(0, CHUNK, unroll=2)
        def body(i):
            for g in range(G):
                x = buf[i, pl.ds(g * 16, 16)]
                tb = (x - lo[g]) * scale[g]
                tb = jnp.minimum(tb, jnp.float32(NBINS - 1))
                idx = tb.astype(jnp.int32)
                within = (x >= lo[g]) & (x <= hi[g])
                plsc.addupdate_scatter(hist, [idx + base[g]], ones,
                                       mask=within)

    _double_buffered(src_hbm, wid, bufs, sems, consume)


def _real_body(xr_hbm, hist_hbm, lo_hbm, hi_hbm, buf0, buf1, hist_r,
               stage_v, sem0, sem1):
    wid = _worker_id()
    iota = lax.iota(jnp.int32, 16)
    base = [(g * 16 + iota) * NBINS for g in range(G)]
    bufs, sems = [buf0, buf1], [sem0, sem1]
    zeros = jnp.zeros((16,), jnp.float32)

    carry0 = tuple(
        tuple(jnp.full((16,), s * jnp.inf, jnp.float32) for _ in range(G))
        for s in (1.0, -1.0, 1.0, -1.0)
    )
    state = [carry0]

    def mm_consume(buf):
        def mmbody(i, carry):
            mn_a, mx_a, mn_b, mx_b = carry
            new_mn_a = tuple(
                jnp.minimum(mn_a[g], buf[i, pl.ds(g * 16, 16)])
                for g in range(G)
            )
            new_mx_a = tuple(
                jnp.maximum(mx_a[g], buf[i, pl.ds(g * 16, 16)])
                for g in range(G)
            )
            new_mn_b = tuple(
                jnp.minimum(mn_b[g], buf[i + 1, pl.ds(g * 16, 16)])
                for g in range(G)
            )
            new_mx_b = tuple(
                jnp.maximum(mx_b[g], buf[i + 1, pl.ds(g * 16, 16)])
                for g in range(G)
            )
            return new_mn_a, new_mx_a, new_mn_b, new_mx_b

        state[0] = plsc.parallel_loop(0, CHUNK, step=2, unroll=2,
                                      carry=state[0])(mmbody)

    _double_buffered(xr_hbm, wid, bufs, sems, mm_consume)
    mn_a, mx_a, mn_b, mx_b = state[0]
    mns = [jnp.minimum(mn_a[g], mn_b[g]) for g in range(G)]
    mxs = [jnp.maximum(mx_a[g], mx_b[g]) for g in range(G)]

    lo, hi, scale = [], [], []
    for g in range(G):
        mn, mx = mns[g], mxs[g]
        same = jnp.abs(mx - mn) < 1e-10
        mx = jnp.where(same, mx + 1e-5, mx)
        mn = jnp.where(same, mn - 1e-5, mn)
        lo.append(mn)
        hi.append(mx)
        scale.append((1.0 / (mx - mn)) * jnp.float32(NBINS))

    for g in range(G):
        stage_v[pl.ds(g * 16, 16)] = lo[g]
    pltpu.sync_copy(stage_v, lo_hbm.at[pl.ds(wid * CPW, CPW)])
    for g in range(G):
        stage_v[pl.ds(g * 16, 16)] = hi[g]
    pltpu.sync_copy(stage_v, hi_hbm.at[pl.ds(wid * CPW, CPW)])

    @plsc.parallel_loop(0, HPW // 16, unroll=4)
    def zbody(i):
        hist_r[pl.ds(i * 16, 16)] = zeros

    _hist_pass(bufs, sems, xr_hbm, wid, hist_r, lo, hi, scale, base)
    pltpu.sync_copy(hist_r, hist_hbm.at[pl.ds(wid * HPW, HPW)])


def _fake_body(xf_hbm, hist_hbm, lo_hbm, hi_hbm, out_hbm, buf0, buf1,
               hist_r, hist_f, stage_v, sem0, sem1):
    wid = _worker_id()
    iota = lax.iota(jnp.int32, 16)
    base = [(g * 16 + iota) * NBINS for g in range(G)]
    bufs, sems = [buf0, buf1], [sem0, sem1]
    zeros = jnp.zeros((16,), jnp.float32)

    hist_copy = pltpu.async_copy(
        hist_hbm.at[pl.ds(wid * HPW, HPW)], hist_r, sem1
    )
    pltpu.sync_copy(lo_hbm.at[pl.ds(wid * CPW, CPW)], stage_v)
    lo = [stage_v[pl.ds(g * 16, 16)] for g in range(G)]
    pltpu.sync_copy(hi_hbm.at[pl.ds(wid * CPW, CPW)], stage_v)
    hi = [stage_v[pl.ds(g * 16, 16)] for g in range(G)]
    scale = [(1.0 / (hi[g] - lo[g])) * jnp.float32(NBINS) for g in range(G)]
    hist_copy.wait()

    @plsc.parallel_loop(0, HPW // 16, unroll=4)
    def zbody(i):
        hist_f[pl.ds(i * 16, 16)] = zeros

    _hist_pass(bufs, sems, xf_hbm, wid, hist_f, lo, hi, scale, base)

    inv_n = jnp.float32(1.0 / N)
    for g in range(G):
        colbase = base[g]

        def fbody(b, carry):
            sa, sf = carry
            cr = plsc.load_gather(hist_r, [colbase + b])
            cf = plsc.load_gather(hist_f, [colbase + b])
            return sa + jnp.abs(cf - cr), sf + cf

        sa, sf = plsc.parallel_loop(0, NBINS, unroll=4, carry=(zeros, zeros))(
            fbody
        )
        loss_g = sa * inv_n + (jnp.float32(N) - sf) * inv_n
        bw = (hi[g] - lo[g]) * jnp.float32(1.0 / NBINS)
        c_first = lo[g] + bw * jnp.float32(0.5)
        c_last = lo[g] + bw * jnp.float32(NBINS - 0.5)
        deg = (jnp.abs(c_first) < 1e-16) & (jnp.abs(c_last) < 1e-16)
        loss_g = jnp.where(deg, jnp.float32(2.0), loss_g)
        stage_v[pl.ds(g * 16, 16)] = loss_g

    pltpu.sync_copy(stage_v, out_hbm.at[pl.ds(wid * CPW, CPW)])


@jax.jit
def _hist_loss(xr, xf):
    f32 = jnp.float32
    hist_hbm, lo_hbm, hi_hbm = pl.kernel(
        _real_body,
        out_type=(
            jax.ShapeDtypeStruct((NW * HPW,), f32),
            jax.ShapeDtypeStruct((NW * CPW,), f32),
            jax.ShapeDtypeStruct((NW * CPW,), f32),
        ),
        mesh=_MESH,
        compiler_params=_PARAMS,
        scratch_types=[
            pltpu.VMEM((CHUNK, CPW), f32),
            pltpu.VMEM((CHUNK, CPW), f32),
            pltpu.VMEM((HPW,), f32),
            pltpu.VMEM((CPW,), f32),
            pltpu.SemaphoreType.DMA,
            pltpu.SemaphoreType.DMA,
        ],
    )(xr)
    return pl.kernel(
        _fake_body,
        out_type=jax.ShapeDtypeStruct((L * D,), f32),
        mesh=_MESH,
        compiler_params=_PARAMS,
        scratch_types=[
            pltpu.VMEM((CHUNK, CPW), f32),
            pltpu.VMEM((CHUNK, CPW), f32),
            pltpu.VMEM((HPW,), f32),
            pltpu.VMEM((HPW,), f32),
            pltpu.VMEM((CPW,), f32),
            pltpu.SemaphoreType.DMA,
            pltpu.SemaphoreType.DMA,
        ],
    )(xf, hist_hbm, lo_hbm, hi_hbm)


def kernel(x_real, x_fake, n_bins):
    del n_bins
    xr = x_real.reshape(N, NW, CPW)
    xf = x_fake.reshape(N, NW, CPW)
    return _hist_loss(xr, xf).reshape(L, D)


# speedup vs baseline: 1.2017x; 1.1217x over previous
import jax
import jax.numpy as jnp
from jax import lax
from jax.experimental import pallas as pl
from jax.experimental.pallas import tpu as pltpu
from jax.experimental.pallas import tpu_sc as plsc

N, L, D, NBINS = 4096, 64, 32, 256
NC, NS = 2, 16
NW = NC * NS
CPW = (L * D) // NW
G = CPW // 16
HPW = CPW * NBINS
CHUNK = 256
NCHUNK = N // CHUNK

_MESH = plsc.VectorSubcoreMesh(
    core_axis_name="c", subcore_axis_name="s", num_cores=NC, num_subcores=NS
)
_PARAMS = pltpu.CompilerParams(needs_layout_passes=False)


def _worker_id():
    return lax.axis_index("c") * NS + lax.axis_index("s")


def _double_buffered(src_hbm, wid, bufs, sems, consume):
    blk = wid // 2
    copies = [None, None]
    copies[0] = pltpu.async_copy(
        src_hbm.at[pl.ds(0, CHUNK), blk], bufs[0], sems[0]
    )
    for ch in range(NCHUNK):
        cur = ch % 2
        copies[cur].wait()
        if ch + 1 < NCHUNK:
            nxt = 1 - cur
            copies[nxt] = pltpu.async_copy(
                src_hbm.at[pl.ds((ch + 1) * CHUNK, CHUNK), blk],
                bufs[nxt],
                sems[nxt],
            )
        consume(bufs[cur])


def _hist_pass(bufs, sems, src_hbm, wid, off, hist, lo, hi, scale, base):
    ones = jnp.ones((16,), jnp.float32)

    def consume(buf):
        @plsc.parallel_loop(0, CHUNK, unroll=2)
        def body(i):
            for g in range(G):
                x = buf[i, pl.ds(off + g * 16, 16)]
                tb = (x - lo[g]) * scale[g]
                tb = jnp.minimum(tb, jnp.float32(NBINS - 1))
                idx = tb.astype(jnp.int32)
                within = (x >= lo[g]) & (x <= hi[g])
                plsc.addupdate_scatter(hist, [idx + base[g]], ones,
                                       mask=within)

    _double_buffered(src_hbm, wid, bufs, sems, consume)


def _real_body(xr_hbm, hist_hbm, lo_hbm, hi_hbm, buf0, buf1, hist_r,
               stage_v, sem0, sem1):
    wid = _worker_id()
    off = (wid % 2) * CPW
    iota = lax.iota(jnp.int32, 16)
    base = [(g * 16 + iota) * NBINS for g in range(G)]
    bufs, sems = [buf0, buf1], [sem0, sem1]
    zeros = jnp.zeros((16,), jnp.float32)

    carry0 = tuple(
        tuple(jnp.full((16,), s * jnp.inf, jnp.float32) for _ in range(G))
        for s in (1.0, -1.0, 1.0, -1.0)
    )
    state = [carry0]

    def mm_consume(buf):
        def mmbody(i, carry):
            mn_a, mx_a, mn_b, mx_b = carry
            new_mn_a = tuple(
                jnp.minimum(mn_a[g], buf[i, pl.ds(off + g * 16, 16)])
                for g in range(G)
            )
            new_mx_a = tuple(
                jnp.maximum(mx_a[g], buf[i, pl.ds(off + g * 16, 16)])
                for g in range(G)
            )
            new_mn_b = tuple(
                jnp.minimum(mn_b[g], buf[i + 1, pl.ds(off + g * 16, 16)])
                for g in range(G)
            )
            new_mx_b = tuple(
                jnp.maximum(mx_b[g], buf[i + 1, pl.ds(off + g * 16, 16)])
                for g in range(G)
            )
            return new_mn_a, new_mx_a, new_mn_b, new_mx_b

        state[0] = plsc.parallel_loop(0, CHUNK, step=2, unroll=2,
                                      carry=state[0])(mmbody)

    _double_buffered(xr_hbm, wid, bufs, sems, mm_consume)
    mn_a, mx_a, mn_b, mx_b = state[0]
    mns = [jnp.minimum(mn_a[g], mn_b[g]) for g in range(G)]
    mxs = [jnp.maximum(mx_a[g], mx_b[g]) for g in range(G)]

    lo, hi, scale = [], [], []
    for g in range(G):
        mn, mx = mns[g], mxs[g]
        same = jnp.abs(mx - mn) < 1e-10
        mx = jnp.where(same, mx + 1e-5, mx)
        mn = jnp.where(same, mn - 1e-5, mn)
        lo.append(mn)
        hi.append(mx)
        scale.append((1.0 / (mx - mn)) * jnp.float32(NBINS))

    for g in range(G):
        stage_v[pl.ds(g * 16, 16)] = lo[g]
    pltpu.sync_copy(stage_v, lo_hbm.at[pl.ds(wid * CPW, CPW)])
    for g in range(G):
        stage_v[pl.ds(g * 16, 16)] = hi[g]
    pltpu.sync_copy(stage_v, hi_hbm.at[pl.ds(wid * CPW, CPW)])

    @plsc.parallel_loop(0, HPW // 16, unroll=4)
    def zbody(i):
        hist_r[pl.ds(i * 16, 16)] = zeros

    _hist_pass(bufs, sems, xr_hbm, wid, off, hist_r, lo, hi, scale, base)
    pltpu.sync_copy(hist_r, hist_hbm.at[pl.ds(wid * HPW, HPW)])


def _fake_body(xf_hbm, hist_hbm, lo_hbm, hi_hbm, out_hbm, buf0, buf1,
               hist_r, hist_f, stage_v, sem0, sem1):
    wid = _worker_id()
    off = (wid % 2) * CPW
    iota = lax.iota(jnp.int32, 16)
    base = [(g * 16 + iota) * NBINS for g in range(G)]
    bufs, sems = [buf0, buf1], [sem0, sem1]
    zeros = jnp.zeros((16,), jnp.float32)

    hist_copy = pltpu.async_copy(
        hist_hbm.at[pl.ds(wid * HPW, HPW)], hist_r, sem1
    )
    pltpu.sync_copy(lo_hbm.at[pl.ds(wid * CPW, CPW)], stage_v)
    lo = [stage_v[pl.ds(g * 16, 16)] for g in range(G)]
    pltpu.sync_copy(hi_hbm.at[pl.ds(wid * CPW, CPW)], stage_v)
    hi = [stage_v[pl.ds(g * 16, 16)] for g in range(G)]
    scale = [(1.0 / (hi[g] - lo[g])) * jnp.float32(NBINS) for g in range(G)]
    hist_copy.wait()

    @plsc.parallel_loop(0, HPW // 16, unroll=4)
    def zbody(i):
        hist_f[pl.ds(i * 16, 16)] = zeros

    _hist_pass(bufs, sems, xf_hbm, wid, off, hist_f, lo, hi, scale, base)

    inv_n = jnp.float32(1.0 / N)
    for g in range(G):
        colbase = base[g]

        def fbody(b, carry):
            sa, sf = carry
            cr = plsc.load_gather(hist_r, [colbase + b])
            cf = plsc.load_gather(hist_f, [colbase + b])
            return sa + jnp.abs(cf - cr), sf + cf

        sa, sf = plsc.parallel_loop(0, NBINS, unroll=4, carry=(zeros, zeros))(
            fbody
        )
        loss_g = sa * inv_n + (jnp.float32(N) - sf) * inv_n
        bw = (hi[g] - lo[g]) * jnp.float32(1.0 / NBINS)
        c_first = lo[g] + bw * jnp.float32(0.5)
        c_last = lo[g] + bw * jnp.float32(NBINS - 0.5)
        deg = (jnp.abs(c_first) < 1e-16) & (jnp.abs(c_last) < 1e-16)
        loss_g = jnp.where(deg, jnp.float32(2.0), loss_g)
        stage_v[pl.ds(g * 16, 16)] = loss_g

    pltpu.sync_copy(stage_v, out_hbm.at[pl.ds(wid * CPW, CPW)])


@jax.jit
def _hist_loss(xr, xf):
    f32 = jnp.float32
    hist_hbm, lo_hbm, hi_hbm = pl.kernel(
        _real_body,
        out_type=(
            jax.ShapeDtypeStruct((NW * HPW,), f32),
            jax.ShapeDtypeStruct((NW * CPW,), f32),
            jax.ShapeDtypeStruct((NW * CPW,), f32),
        ),
        mesh=_MESH,
        compiler_params=_PARAMS,
        scratch_types=[
            pltpu.VMEM((CHUNK, 2 * CPW), f32),
            pltpu.VMEM((CHUNK, 2 * CPW), f32),
            pltpu.VMEM((HPW,), f32),
            pltpu.VMEM((CPW,), f32),
            pltpu.SemaphoreType.DMA,
            pltpu.SemaphoreType.DMA,
        ],
    )(xr)
    return pl.kernel(
        _fake_body,
        out_type=jax.ShapeDtypeStruct((L * D,), f32),
        mesh=_MESH,
        compiler_params=_PARAMS,
        scratch_types=[
            pltpu.VMEM((CHUNK, 2 * CPW), f32),
            pltpu.VMEM((CHUNK, 2 * CPW), f32),
            pltpu.VMEM((HPW,), f32),
            pltpu.VMEM((HPW,), f32),
            pltpu.VMEM((CPW,), f32),
            pltpu.SemaphoreType.DMA,
            pltpu.SemaphoreType.DMA,
        ],
    )(xf, hist_hbm, lo_hbm, hi_hbm)


def kernel(x_real, x_fake, n_bins):
    del n_bins
    xr = x_real.reshape(N, NW // 2, 2 * CPW)
    xf = x_fake.reshape(N, NW // 2, 2 * CPW)
    return _hist_loss(xr, xf).reshape(L, D)
